# Initial kernel scaffold; baseline (speedup 1.0000x reference)
import jax
import jax.numpy as jnp
from jax.experimental import pallas as pl


def kernel(input_ids, attention_mask, labels, weight):
    raise NotImplementedError("write your pallas kernel here")


# speedup vs baseline: 1.7466x; 1.7466x over previous
import functools

import jax
import jax.numpy as jnp
from jax import lax
from jax.experimental import pallas as pl
from jax.experimental.pallas import tpu as pltpu
from jax.experimental.pallas import tpu_sc as plsc

VOCAB = 32000
D_MODEL = 2048
B = 4
S = 4096

NUM_TOKENS = B * S
NC = 2
NS = 16
NW = NC * NS
TOK_PER_W = NUM_TOKENS // NW
CHUNK = 16
NCHUNK = TOK_PER_W // CHUNK
NBUF = 2


def _gather_kernel(ids_hbm, table_hbm, out_hbm, idx_v, buf0, buf1,
                   gsem0, gsem1, ssem0, ssem1):
  wid = lax.axis_index("s") * NC + lax.axis_index("c")
  base = wid * TOK_PER_W

  bufs = (buf0, buf1)
  gsems = (gsem0, gsem1)
  ssems = (ssem0, ssem1)

  pltpu.sync_copy(ids_hbm.at[wid], idx_v)

  def out_slice(i):
    return out_hbm.at[pl.ds(base + i * CHUNK, CHUNK), :]

  for b in range(NBUF):
    pltpu.async_copy(table_hbm.at[idx_v.at[b]], bufs[b], gsems[b])

  def body(g, carry):
    for b in range(NBUF):
      i = g * NBUF + b
      pltpu.make_async_copy(table_hbm.at[idx_v.at[i]], bufs[b],
                            gsems[b]).wait()
      pltpu.async_copy(bufs[b], out_slice(i), ssems[b])
      pltpu.make_async_copy(bufs[b], out_slice(i), ssems[b]).wait()
      nxt = i + NBUF

      @pl.when(nxt < NCHUNK)
      def _():
        pltpu.async_copy(table_hbm.at[idx_v.at[nxt]], bufs[b], gsems[b])

    return carry

  lax.fori_loop(0, NCHUNK // NBUF, body, 0)


@jax.jit
def _embed(ids_flat, weight):
  mesh = plsc.VectorSubcoreMesh(core_axis_name="c", subcore_axis_name="s")
  k = functools.partial(
      pl.kernel,
      mesh=mesh,
      out_type=jax.ShapeDtypeStruct((NUM_TOKENS, D_MODEL), jnp.float32),
      scratch_types=[
          pltpu.VMEM((NCHUNK, CHUNK), jnp.int32),
          pltpu.VMEM((CHUNK, D_MODEL), jnp.float32),
          pltpu.VMEM((CHUNK, D_MODEL), jnp.float32),
          pltpu.SemaphoreType.DMA,
          pltpu.SemaphoreType.DMA,
          pltpu.SemaphoreType.DMA,
          pltpu.SemaphoreType.DMA,
      ],
  )(_gather_kernel)
  ids3 = ids_flat.reshape(NW, NCHUNK, CHUNK)
  return k(ids3, weight)


def kernel(input_ids, attention_mask, labels, weight):
  batch_size, seq_length = input_ids.shape
  position_ids = jnp.arange(seq_length, dtype=jnp.int32)[None, :]
  ids_flat = input_ids.astype(jnp.int32).reshape(-1)
  hidden = _embed(ids_flat, weight).reshape(batch_size, seq_length, D_MODEL)
  return (hidden, attention_mask, position_ids, labels)
